# baseline (speedup 1.0000x reference)
import functools

import jax
import jax.numpy as jnp
import numpy as np
from jax import lax
from jax.experimental import pallas as pl
from jax.experimental.pallas import tpu as pltpu
from jax.experimental.pallas import tpu_sc as plsc

B = 1024
D_IN = 256
D_H = 512
D = 128
N_KEYS = 100000
N_CLASSES = 16
K_NN = 10
N_HEADS = 4
DH = D // N_HEADS

B_TILE = 1024
KEY_BLK = 4000
N_KB = N_KEYS // KEY_BLK
RUN_W = 16
GR = 64
NG = B_TILE // GR


def _enc_body(x_ref, w1_ref, b1_ref, w2_ref, b2_ref, z_ref):
    h = jnp.dot(x_ref[...], w1_ref[...], preferred_element_type=jnp.float32)
    h = jnp.maximum(h + b1_ref[...], 0.0)
    z = jnp.dot(h, w2_ref[...], preferred_element_type=jnp.float32) + b2_ref[...]
    ss = jnp.sum(z * z, axis=1, keepdims=True)
    z_ref[...] = z * (1.0 / (jnp.sqrt(ss) + 1e-12))


def _encoder(x, W1, b1, W2, b2):
    return pl.pallas_call(
        _enc_body,
        out_shape=jax.ShapeDtypeStruct((B, D), jnp.float32),
    )(x, W1, b1.reshape(1, D_H), W2, b2.reshape(1, D))


def _knn_body(z_ref, keys_ref, sims_ref, idx_ref, rv_ref, ri_ref, thv_ref,
              s_ref):
    ph = pl.program_id(0)
    kb = pl.program_id(1)

    @pl.when((ph == 0) & (kb == 0))
    def _():
        thv_ref[...] = jnp.full((B_TILE, RUN_W), -jnp.inf, jnp.float32)

    @pl.when((ph == 1) & (kb == 0))
    def _():
        rv_ref[...] = jnp.full((B_TILE, RUN_W), -jnp.inf, jnp.float32)
        ri_ref[...] = jnp.zeros((B_TILE, RUN_W), jnp.int32)

    keys = keys_ref[...]
    ss = jnp.sum(keys * keys, axis=1, keepdims=True)
    keys_n = keys * (1.0 / (jnp.sqrt(ss) + 1e-12))
    s = lax.dot_general(z_ref[...], keys_n, (((1,), (1,)), ((), ())),
                        preferred_element_type=jnp.float32)

    lane_rf = lax.broadcasted_iota(jnp.int32, (B_TILE, RUN_W), 1)

    @pl.when(ph == 0)
    def _():
        bm = jnp.max(s, axis=1, keepdims=True)
        tv = thv_ref[...]
        ge = tv >= bm
        tv_s = jnp.concatenate(
            [jnp.full((B_TILE, 1), jnp.inf, jnp.float32), tv[:, :RUN_W - 1]],
            axis=1)
        ge_s = tv_s >= bm
        ntv = jnp.where(ge, tv, jnp.where(
            ge_s, jnp.broadcast_to(bm, (B_TILE, RUN_W)), tv_s))
        thv_ref[...] = jnp.where(lane_rf >= K_NN, -jnp.inf, ntv)

    @pl.when(ph == 1)
    def _():
        th = thv_ref[:, K_NN - 1:K_NN]
        s_ref[...] = s
        qf = (s >= th).astype(jnp.float32)
        ones_col = jnp.full((KEY_BLK, 1), 1.0, jnp.float32)
        cnt = jnp.dot(qf, ones_col, preferred_element_type=jnp.float32)

        lane_b = lax.broadcasted_iota(jnp.int32, (GR, KEY_BLK), 1)
        lane_r = lax.broadcasted_iota(jnp.int32, (GR, RUN_W), 1)
        base = kb * KEY_BLK

        for g in range(NG):
            rows = pl.ds(g * GR, GR)
            t_g = jnp.minimum(
                jnp.max(cnt[g * GR:(g + 1) * GR, :]).astype(jnp.int32), K_NN)

            def ins_body(t, _):
                sg = s_ref[rows, :]
                m = jnp.max(sg, axis=1, keepdims=True)
                p = jnp.min(jnp.where(sg == m, lane_b, KEY_BLK), axis=1,
                            keepdims=True)
                gi = base + p
                s_ref[rows, :] = jnp.where(lane_b == p, -jnp.inf, sg)
                rv = rv_ref[rows, :]
                ri = ri_ref[rows, :]
                ge = rv >= m
                rv_s = jnp.concatenate(
                    [jnp.full((GR, 1), jnp.inf, jnp.float32),
                     rv[:, :RUN_W - 1]], axis=1)
                ri_s = jnp.concatenate(
                    [jnp.zeros((GR, 1), jnp.int32), ri[:, :RUN_W - 1]],
                    axis=1)
                ge_s = rv_s >= m
                nrv = jnp.where(ge, rv, jnp.where(
                    ge_s, jnp.broadcast_to(m, (GR, RUN_W)), rv_s))
                nri = jnp.where(ge, ri, jnp.where(
                    ge_s, jnp.broadcast_to(gi, (GR, RUN_W)), ri_s))
                rv_ref[rows, :] = jnp.where(lane_r >= K_NN, -jnp.inf, nrv)
                ri_ref[rows, :] = nri
                return 0

            lax.fori_loop(0, t_g, ins_body, 0)

    @pl.when((ph == 1) & (kb == N_KB - 1))
    def _():
        sims_ref[...] = rv_ref[:, :K_NN]
        idx_ref[...] = ri_ref[:, :K_NN]


def _knn(z, index_keys):
    return pl.pallas_call(
        _knn_body,
        grid=(2, N_KB),
        in_specs=[
            pl.BlockSpec((B_TILE, D), lambda ph, kb: (0, 0)),
            pl.BlockSpec((KEY_BLK, D), lambda ph, kb: (kb, 0)),
        ],
        out_specs=[
            pl.BlockSpec((B_TILE, K_NN), lambda ph, kb: (0, 0)),
            pl.BlockSpec((B_TILE, K_NN), lambda ph, kb: (0, 0)),
        ],
        out_shape=[
            jax.ShapeDtypeStruct((B, K_NN), jnp.float32),
            jax.ShapeDtypeStruct((B, K_NN), jnp.int32),
        ],
        scratch_shapes=[
            pltpu.VMEM((B_TILE, RUN_W), jnp.float32),
            pltpu.VMEM((B_TILE, RUN_W), jnp.int32),
            pltpu.VMEM((B_TILE, RUN_W), jnp.float32),
            pltpu.VMEM((B_TILE, KEY_BLK), jnp.float32),
        ],
        compiler_params=pltpu.CompilerParams(
            dimension_semantics=("arbitrary", "arbitrary")),
    )(z, index_keys)


_NW = 32
_BPW = (B * K_NN) // _NW


def _sc_gather(index_keys, labels_pad, idx_flat, rowidx_flat):
    mesh = plsc.VectorSubcoreMesh(core_axis_name="c", subcore_axis_name="s")

    @functools.partial(
        pl.kernel,
        mesh=mesh,
        out_type=(
            jax.ShapeDtypeStruct((B * K_NN, D), jnp.float32),
            jax.ShapeDtypeStruct((B * K_NN, 128), jnp.int32),
        ),
        scratch_types=[
            pltpu.VMEM((_BPW,), jnp.int32),
            pltpu.VMEM((_BPW,), jnp.int32),
            pltpu.VMEM((_BPW, D), jnp.float32),
            pltpu.VMEM((_BPW, 128), jnp.int32),
            pltpu.SemaphoreType.DMA,
            pltpu.SemaphoreType.DMA,
        ],
    )
    def k(keys_hbm, lab_hbm, idx_hbm, ridx_hbm, rows_out, labs_out, idx_v,
          ridx_v, rows_v, labs_v, sem1, sem2):
        wid = lax.axis_index("s") * 2 + lax.axis_index("c")
        base = wid * _BPW
        pltpu.sync_copy(idx_hbm.at[pl.ds(base, _BPW)], idx_v)
        pltpu.sync_copy(ridx_hbm.at[pl.ds(base, _BPW)], ridx_v)
        cp1 = pltpu.async_copy(keys_hbm.at[idx_v], rows_v, sem1)
        cp2 = pltpu.async_copy(lab_hbm.at[ridx_v], labs_v, sem2)
        cp1.wait()
        cp2.wait()
        pltpu.sync_copy(rows_v, rows_out.at[pl.ds(base, _BPW)])
        pltpu.sync_copy(labs_v, labs_out.at[pl.ds(base, _BPW)])

    return k(index_keys, labels_pad, idx_flat, rowidx_flat)


AT = 512
ATK = AT * K_NN


def _attn_body(z_ref, rows_ref, labs_ref, idxf_ref, le_ref, wq_ref, wk_ref,
               wv_ref, wo_ref, wh1_ref, bh1_ref, wh2_ref, bh2_ref, out_ref,
               nlab_ref):
    rows = rows_ref[...]
    ss = jnp.sum(rows * rows, axis=1, keepdims=True)
    rows_n = rows * (1.0 / (jnp.sqrt(ss) + 1e-12))
    lane128 = lax.broadcasted_iota(jnp.int32, (ATK, 128), 1)
    col = jnp.bitwise_and(idxf_ref[...], 127)
    lab = jnp.sum(jnp.where(lane128 == col, labs_ref[...], 0), axis=1,
                  keepdims=True)
    nlab_ref[...] = lab
    cls = lax.broadcasted_iota(jnp.int32, (ATK, N_CLASSES), 1)
    onehot = (lab == cls).astype(jnp.float32)
    kv = rows_n + jnp.dot(onehot, le_ref[...], preferred_element_type=jnp.float32)
    kk = jnp.dot(kv, wk_ref[...], preferred_element_type=jnp.float32)
    vv = jnp.dot(kv, wv_ref[...], preferred_element_type=jnp.float32)
    q = jnp.dot(z_ref[...], wq_ref[...], preferred_element_type=jnp.float32)
    kk3 = kk.reshape(AT, K_NN, D)
    vv3 = vv.reshape(AT, K_NN, D)
    p3 = q[:, None, :] * kk3
    scale = 1.0 / np.sqrt(DH)
    scores = jnp.concatenate(
        [jnp.sum(p3[:, :, h * DH:(h + 1) * DH], axis=2, keepdims=True)
         for h in range(N_HEADS)], axis=2) * scale
    mx = jnp.max(scores, axis=1, keepdims=True)
    e = jnp.exp(scores - mx)
    a = e / jnp.sum(e, axis=1, keepdims=True)
    ae = jnp.concatenate(
        [jnp.broadcast_to(a[:, :, h:h + 1], (AT, K_NN, DH))
         for h in range(N_HEADS)], axis=2)
    ctx = jnp.sum(ae * vv3, axis=1)
    o = jnp.dot(ctx, wo_ref[...], preferred_element_type=jnp.float32)
    hh = jnp.maximum(
        jnp.dot(o, wh1_ref[...], preferred_element_type=jnp.float32)
        + bh1_ref[...], 0.0)
    out_ref[...] = (jnp.dot(hh, wh2_ref[...], preferred_element_type=jnp.float32)
                    + bh2_ref[...])


def _attention(z, rows, labs, idxf, label_emb, Wq, Wk, Wv, Wo, Wh1, bh1,
               Wh2, bh2):
    full = lambda r, c: pl.BlockSpec((r, c), lambda i: (0, 0))
    return pl.pallas_call(
        _attn_body,
        grid=(B // AT,),
        in_specs=[
            pl.BlockSpec((AT, D), lambda i: (i, 0)),
            pl.BlockSpec((ATK, D), lambda i: (i, 0)),
            pl.BlockSpec((ATK, 128), lambda i: (i, 0)),
            pl.BlockSpec((ATK, 1), lambda i: (i, 0)),
            full(N_CLASSES, D), full(D, D), full(D, D), full(D, D),
            full(D, D), full(D, 2 * D), full(1, 2 * D), full(2 * D, N_CLASSES),
            full(1, N_CLASSES),
        ],
        out_specs=[
            pl.BlockSpec((AT, N_CLASSES), lambda i: (i, 0)),
            pl.BlockSpec((ATK, 1), lambda i: (i, 0)),
        ],
        out_shape=[
            jax.ShapeDtypeStruct((B, N_CLASSES), jnp.float32),
            jax.ShapeDtypeStruct((B * K_NN, 1), jnp.int32),
        ],
    )(z, rows, labs, idxf, label_emb, Wq, Wk, Wv, Wo, Wh1,
      bh1.reshape(1, 2 * D), Wh2, bh2.reshape(1, N_CLASSES))


def kernel(x, index_keys, index_labels, W1, b1, W2, b2, label_emb, Wq, Wk,
           Wv, Wo, Wh1, bh1, Wh2, bh2):
    labels_i32 = index_labels.astype(jnp.int32)
    z = _encoder(x, W1, b1, W2, b2)
    sims, idx = _knn(z, index_keys)
    labels_pad = jnp.concatenate(
        [labels_i32, jnp.zeros((800 * 128 - N_KEYS,), jnp.int32)]).reshape(800, 128)
    idx_flat = idx.reshape(-1)
    rows, labs = _sc_gather(index_keys, labels_pad, idx_flat,
                            idx_flat // 128)
    logits, nlab = _attention(z, rows, labs, idx_flat.reshape(-1, 1),
                              label_emb, Wq, Wk, Wv, Wo, Wh1, bh1, Wh2, bh2)
    n_labels = nlab.reshape(B, K_NN)
    return (logits, z, idx, n_labels, sims)


# speedup vs baseline: 1.0482x; 1.0482x over previous
import functools

import jax
import jax.numpy as jnp
import numpy as np
from jax import lax
from jax.experimental import pallas as pl
from jax.experimental.pallas import tpu as pltpu
from jax.experimental.pallas import tpu_sc as plsc

B = 1024
D_IN = 256
D_H = 512
D = 128
N_KEYS = 100000
N_CLASSES = 16
K_NN = 10
N_HEADS = 4
DH = D // N_HEADS

B_TILE = 1024
KEY_BLK = 4000
N_KB = N_KEYS // KEY_BLK
RUN_W = 16
GR = 64
NG = B_TILE // GR


def _enc_body(x_ref, w1_ref, b1_ref, w2_ref, b2_ref, z_ref):
    h = jnp.dot(x_ref[...], w1_ref[...], preferred_element_type=jnp.float32)
    h = jnp.maximum(h + b1_ref[...], 0.0)
    z = jnp.dot(h, w2_ref[...], preferred_element_type=jnp.float32) + b2_ref[...]
    ss = jnp.sum(z * z, axis=1, keepdims=True)
    z_ref[...] = z * (1.0 / (jnp.sqrt(ss) + 1e-12))


def _encoder(x, W1, b1, W2, b2):
    return pl.pallas_call(
        _enc_body,
        out_shape=jax.ShapeDtypeStruct((B, D), jnp.float32),
    )(x, W1, b1.reshape(1, D_H), W2, b2.reshape(1, D))


def _knn_body(z_ref, keys_ref, sims_ref, idx_ref, rv_ref, ri_ref, thv_ref,
              s_ref):
    ph = pl.program_id(0)
    kb = pl.program_id(1)

    @pl.when((ph == 0) & (kb == 0))
    def _():
        thv_ref[...] = jnp.full((B_TILE, RUN_W), -jnp.inf, jnp.float32)

    @pl.when((ph == 1) & (kb == 0))
    def _():
        rv_ref[...] = jnp.full((B_TILE, RUN_W), -jnp.inf, jnp.float32)
        ri_ref[...] = jnp.zeros((B_TILE, RUN_W), jnp.int32)

    keys = keys_ref[...]
    ss = jnp.sum(keys * keys, axis=1, keepdims=True)
    keys_n = keys * (1.0 / (jnp.sqrt(ss) + 1e-12))
    s = lax.dot_general(z_ref[...], keys_n, (((1,), (1,)), ((), ())),
                        preferred_element_type=jnp.float32)

    lane_rf = lax.broadcasted_iota(jnp.int32, (B_TILE, RUN_W), 1)

    @pl.when(ph == 0)
    def _():
        bm = jnp.max(s, axis=1, keepdims=True)
        tv = thv_ref[...]
        ge = tv >= bm
        tv_s = jnp.concatenate(
            [jnp.full((B_TILE, 1), jnp.inf, jnp.float32), tv[:, :RUN_W - 1]],
            axis=1)
        ge_s = tv_s >= bm
        ntv = jnp.where(ge, tv, jnp.where(
            ge_s, jnp.broadcast_to(bm, (B_TILE, RUN_W)), tv_s))
        thv_ref[...] = jnp.where(lane_rf >= K_NN, -jnp.inf, ntv)

    @pl.when(ph == 1)
    def _():
        th = thv_ref[:, K_NN - 1:K_NN]
        s_ref[...] = s
        cnt = jnp.sum((s >= th).astype(jnp.int32), axis=1, keepdims=True)

        lane_b = lax.broadcasted_iota(jnp.int32, (GR, KEY_BLK), 1)
        lane_r = lax.broadcasted_iota(jnp.int32, (GR, RUN_W), 1)
        base = kb * KEY_BLK

        for g in range(NG):
            rows = pl.ds(g * GR, GR)
            t_g = jnp.minimum(jnp.max(cnt[g * GR:(g + 1) * GR, :]), K_NN)

            def ins_body(t, _):
                sg = s_ref[rows, :]
                m = jnp.max(sg, axis=1, keepdims=True)
                p = jnp.min(jnp.where(sg == m, lane_b, KEY_BLK), axis=1,
                            keepdims=True)
                gi = base + p
                s_ref[rows, :] = jnp.where(lane_b == p, -jnp.inf, sg)
                rv = rv_ref[rows, :]
                ri = ri_ref[rows, :]
                ge = rv >= m
                rv_s = jnp.concatenate(
                    [jnp.full((GR, 1), jnp.inf, jnp.float32),
                     rv[:, :RUN_W - 1]], axis=1)
                ri_s = jnp.concatenate(
                    [jnp.zeros((GR, 1), jnp.int32), ri[:, :RUN_W - 1]],
                    axis=1)
                ge_s = rv_s >= m
                nrv = jnp.where(ge, rv, jnp.where(
                    ge_s, jnp.broadcast_to(m, (GR, RUN_W)), rv_s))
                nri = jnp.where(ge, ri, jnp.where(
                    ge_s, jnp.broadcast_to(gi, (GR, RUN_W)), ri_s))
                rv_ref[rows, :] = jnp.where(lane_r >= K_NN, -jnp.inf, nrv)
                ri_ref[rows, :] = nri
                return 0

            lax.fori_loop(0, t_g, ins_body, 0)

    @pl.when((ph == 1) & (kb == N_KB - 1))
    def _():
        sims_ref[...] = rv_ref[:, :K_NN]
        idx_ref[...] = ri_ref[:, :K_NN]


def _knn(z, index_keys):
    return pl.pallas_call(
        _knn_body,
        grid=(2, N_KB),
        in_specs=[
            pl.BlockSpec((B_TILE, D), lambda ph, kb: (0, 0)),
            pl.BlockSpec((KEY_BLK, D), lambda ph, kb: (kb, 0)),
        ],
        out_specs=[
            pl.BlockSpec((B_TILE, K_NN), lambda ph, kb: (0, 0)),
            pl.BlockSpec((B_TILE, K_NN), lambda ph, kb: (0, 0)),
        ],
        out_shape=[
            jax.ShapeDtypeStruct((B, K_NN), jnp.float32),
            jax.ShapeDtypeStruct((B, K_NN), jnp.int32),
        ],
        scratch_shapes=[
            pltpu.VMEM((B_TILE, RUN_W), jnp.float32),
            pltpu.VMEM((B_TILE, RUN_W), jnp.int32),
            pltpu.VMEM((B_TILE, RUN_W), jnp.float32),
            pltpu.VMEM((B_TILE, KEY_BLK), jnp.float32),
        ],
        compiler_params=pltpu.CompilerParams(
            dimension_semantics=("arbitrary", "arbitrary")),
    )(z, index_keys)


_NW = 32
_BPW = (B * K_NN) // _NW


def _sc_gather(index_keys, labels_pad, idx_flat, rowidx_flat):
    mesh = plsc.VectorSubcoreMesh(core_axis_name="c", subcore_axis_name="s")

    @functools.partial(
        pl.kernel,
        mesh=mesh,
        out_type=(
            jax.ShapeDtypeStruct((B * K_NN, D), jnp.float32),
            jax.ShapeDtypeStruct((B * K_NN, 128), jnp.int32),
        ),
        scratch_types=[
            pltpu.VMEM((_BPW,), jnp.int32),
            pltpu.VMEM((_BPW,), jnp.int32),
            pltpu.VMEM((_BPW, D), jnp.float32),
            pltpu.VMEM((_BPW, 128), jnp.int32),
            pltpu.SemaphoreType.DMA,
            pltpu.SemaphoreType.DMA,
        ],
    )
    def k(keys_hbm, lab_hbm, idx_hbm, ridx_hbm, rows_out, labs_out, idx_v,
          ridx_v, rows_v, labs_v, sem1, sem2):
        wid = lax.axis_index("s") * 2 + lax.axis_index("c")
        base = wid * _BPW
        pltpu.sync_copy(idx_hbm.at[pl.ds(base, _BPW)], idx_v)
        pltpu.sync_copy(ridx_hbm.at[pl.ds(base, _BPW)], ridx_v)
        cp1 = pltpu.async_copy(keys_hbm.at[idx_v], rows_v, sem1)
        cp2 = pltpu.async_copy(lab_hbm.at[ridx_v], labs_v, sem2)
        cp1.wait()
        cp2.wait()
        pltpu.sync_copy(rows_v, rows_out.at[pl.ds(base, _BPW)])
        pltpu.sync_copy(labs_v, labs_out.at[pl.ds(base, _BPW)])

    return k(index_keys, labels_pad, idx_flat, rowidx_flat)


AT = 512
ATK = AT * K_NN


def _attn_body(z_ref, rows_ref, labs_ref, idxf_ref, le_ref, wq_ref, wk_ref,
               wv_ref, wo_ref, wh1_ref, bh1_ref, wh2_ref, bh2_ref, out_ref,
               nlab_ref):
    rows = rows_ref[...]
    ss = jnp.sum(rows * rows, axis=1, keepdims=True)
    rows_n = rows * (1.0 / (jnp.sqrt(ss) + 1e-12))
    lane128 = lax.broadcasted_iota(jnp.int32, (ATK, 128), 1)
    col = jnp.bitwise_and(idxf_ref[...], 127)
    lab = jnp.sum(jnp.where(lane128 == col, labs_ref[...], 0), axis=1,
                  keepdims=True)
    nlab_ref[...] = lab
    cls = lax.broadcasted_iota(jnp.int32, (ATK, N_CLASSES), 1)
    onehot = (lab == cls).astype(jnp.float32)
    kv = rows_n + jnp.dot(onehot, le_ref[...], preferred_element_type=jnp.float32)
    kk = jnp.dot(kv, wk_ref[...], preferred_element_type=jnp.float32)
    vv = jnp.dot(kv, wv_ref[...], preferred_element_type=jnp.float32)
    q = jnp.dot(z_ref[...], wq_ref[...], preferred_element_type=jnp.float32)
    kk3 = kk.reshape(AT, K_NN, D)
    vv3 = vv.reshape(AT, K_NN, D)
    p3 = q[:, None, :] * kk3
    scale = 1.0 / np.sqrt(DH)
    scores = jnp.concatenate(
        [jnp.sum(p3[:, :, h * DH:(h + 1) * DH], axis=2, keepdims=True)
         for h in range(N_HEADS)], axis=2) * scale
    mx = jnp.max(scores, axis=1, keepdims=True)
    e = jnp.exp(scores - mx)
    a = e / jnp.sum(e, axis=1, keepdims=True)
    ae = jnp.concatenate(
        [jnp.broadcast_to(a[:, :, h:h + 1], (AT, K_NN, DH))
         for h in range(N_HEADS)], axis=2)
    ctx = jnp.sum(ae * vv3, axis=1)
    o = jnp.dot(ctx, wo_ref[...], preferred_element_type=jnp.float32)
    hh = jnp.maximum(
        jnp.dot(o, wh1_ref[...], preferred_element_type=jnp.float32)
        + bh1_ref[...], 0.0)
    out_ref[...] = (jnp.dot(hh, wh2_ref[...], preferred_element_type=jnp.float32)
                    + bh2_ref[...])


def _attention(z, rows, labs, idxf, label_emb, Wq, Wk, Wv, Wo, Wh1, bh1,
               Wh2, bh2):
    full = lambda r, c: pl.BlockSpec((r, c), lambda i: (0, 0))
    return pl.pallas_call(
        _attn_body,
        grid=(B // AT,),
        in_specs=[
            pl.BlockSpec((AT, D), lambda i: (i, 0)),
            pl.BlockSpec((ATK, D), lambda i: (i, 0)),
            pl.BlockSpec((ATK, 128), lambda i: (i, 0)),
            pl.BlockSpec((ATK, 1), lambda i: (i, 0)),
            full(N_CLASSES, D), full(D, D), full(D, D), full(D, D),
            full(D, D), full(D, 2 * D), full(1, 2 * D), full(2 * D, N_CLASSES),
            full(1, N_CLASSES),
        ],
        out_specs=[
            pl.BlockSpec((AT, N_CLASSES), lambda i: (i, 0)),
            pl.BlockSpec((ATK, 1), lambda i: (i, 0)),
        ],
        out_shape=[
            jax.ShapeDtypeStruct((B, N_CLASSES), jnp.float32),
            jax.ShapeDtypeStruct((B * K_NN, 1), jnp.int32),
        ],
    )(z, rows, labs, idxf, label_emb, Wq, Wk, Wv, Wo, Wh1,
      bh1.reshape(1, 2 * D), Wh2, bh2.reshape(1, N_CLASSES))


def kernel(x, index_keys, index_labels, W1, b1, W2, b2, label_emb, Wq, Wk,
           Wv, Wo, Wh1, bh1, Wh2, bh2):
    labels_i32 = index_labels.astype(jnp.int32)
    z = _encoder(x, W1, b1, W2, b2)
    sims, idx = _knn(z, index_keys)
    labels_pad = jnp.concatenate(
        [labels_i32, jnp.zeros((800 * 128 - N_KEYS,), jnp.int32)]).reshape(800, 128)
    idx_flat = idx.reshape(-1)
    rows, labs = _sc_gather(index_keys, labels_pad, idx_flat,
                            idx_flat // 128)
    logits, nlab = _attention(z, rows, labs, idx_flat.reshape(-1, 1),
                              label_emb, Wq, Wk, Wv, Wo, Wh1, bh1, Wh2, bh2)
    n_labels = nlab.reshape(B, K_NN)
    return (logits, z, idx, n_labels, sims)
